# baseline (speedup 1.0000x reference)
import functools

import jax
import jax.numpy as jnp
from jax import lax
from jax.experimental import pallas as pl
from jax.experimental.pallas import tpu as pltpu
from jax.experimental.pallas import tpu_sc as plsc

N_NODES = 10000
N_EDGES = 160000
C = 256

NC = 2
NS = 16
NW = NC * NS

CP = 128
CH = 128
RING = 4

SEG = 4
E_SEG = N_EDGES // SEG
E_PAD_SEG = 40960
G = 2
RING2 = 2
NCHS = 2 * E_PAD_SEG // (CH * G) // NW
FAST_C = 0
NF = 14
NSL = 6
IDX_PAD = 16 * G * CH

BLK_N = 1000
BLK_E = 2000


def _proj_body(n_ref, w_ref, o_ref):
    p = lax.dot_general(n_ref[...], w_ref[...],
                        (((1,), (1,)), ((), ())),
                        preferred_element_type=jnp.float32)
    u = lax.bitcast_convert_type(p, jnp.uint32)
    r = u + jnp.uint32(0x7FFF) + ((u >> 16) & jnp.uint32(1))
    h = r >> 16
    packed = h[:, :CP] | (h[:, CP:] << 16)
    o_ref[...] = lax.bitcast_convert_type(packed, jnp.float32)


def _node_proj(node, W1_one):
    return pl.pallas_call(
        _proj_body,
        grid=(N_NODES // BLK_N,),
        in_specs=[
            pl.BlockSpec((BLK_N, C), lambda i: (i, 0)),
            pl.BlockSpec((C, C), lambda i: (0, 0)),
        ],
        out_specs=pl.BlockSpec((BLK_N, CP), lambda i: (i, 0)),
        out_shape=jax.ShapeDtypeStruct((N_NODES, CP), jnp.float32),
    )(node, W1_one)


def _sc_gather_body(table, idx2, out_s, out_d, idx_v, rows,
                    sg0, sg1, sw0, sw1):
    sg = [sg0, sg1]
    sw = [sw0, sw1]
    c = lax.axis_index("c")
    s = lax.axis_index("s")
    wid = s * NC + c
    is_src = wid < NS
    lw = jnp.where(is_src, wid, wid - NS)

    dps = G * CH
    nda = E_PAD_SEG // dps
    nd = jnp.where(c == FAST_C, NF, NSL)
    nd_even = NF if FAST_C == 0 else NSL
    pair = lw // 2
    base_arr = pair * (NF + NSL) + jnp.where(c == 0, 0, nd_even)
    desc0 = jnp.where(is_src, base_arr, nda + base_arr)

    pltpu.sync_copy(idx2.at[pl.ds(pl.multiple_of(desc0 * dps, 8), NF * dps)],
                    idx_v)

    def gcopy(t, b):
        return pltpu.make_async_copy(
            table.at[idx_v.at[pl.ds(t * dps, dps)]], rows.at[b], sg[b])

    def run(oref):
        def wcopy(t, b):
            return pltpu.make_async_copy(rows.at[b], oref.at[base_arr + t],
                                         sw[b])

        for b in range(RING2):
            gcopy(b, b).start()

        def outer(r, carry):
            t0 = r * RING2
            for b in range(RING2):
                t = t0 + b
                gcopy(t, b).wait()
                wcopy(t, b).start()
                pb = (b - 1) % RING2
                tn = t + RING2 - 1

                @pl.when(jnp.logical_and(t >= 1, tn < nd))
                def _():
                    wcopy(t - 1, pb).wait()
                    gcopy(tn, pb).start()
            return carry

        lax.fori_loop(0, nd // RING2, outer, 0)
        for b in range(RING2):
            wcopy(nd - RING2 + b, b).wait()

    @pl.when(is_src)
    def _():
        run(out_s)

    @pl.when(jnp.logical_not(is_src))
    def _():
        run(out_d)


@functools.cache
def _sc_gather():
    sr = E_PAD_SEG // (G * CH)
    return pl.kernel(
        _sc_gather_body,
        out_type=(jax.ShapeDtypeStruct((sr, G * CH, CP), jnp.float32),
                  jax.ShapeDtypeStruct((sr, G * CH, CP), jnp.float32)),
        mesh=plsc.VectorSubcoreMesh(core_axis_name="c", subcore_axis_name="s",
                                    num_cores=NC, num_subcores=NS),
        scratch_types=[
            pltpu.VMEM((NF * G * CH,), jnp.int32),
            pltpu.VMEM((RING2, G * CH, CP), jnp.float32),
        ] + [pltpu.SemaphoreType.DMA] * 4,
    )


def _main_body(canvas_ref, ps_ref, pd_ref, e_ref, w2o_ref, w1t_ref, w2t_ref,
               b1o_ref, b1t_ref, b2o_ref, b2t_ref,
               g_ref, w1g_ref, b1g_ref, w2g_ref, b2g_ref, o_ref):
    del canvas_ref
    dot = lambda a, b: lax.dot_general(a, b, (((1,), (1,)), ((), ())),
                                       preferred_element_type=jnp.float32)

    def unpack(ref):
        w = lax.bitcast_convert_type(ref[...], jnp.uint32)
        lo = lax.bitcast_convert_type(w << 16, jnp.float32)
        hi = lax.bitcast_convert_type(w & jnp.uint32(0xFFFF0000), jnp.float32)
        return lo, hi

    bf = jnp.bfloat16
    e = e_ref[...]
    psa, psb = unpack(ps_ref)
    pda, pdb = unpack(pd_ref)
    b1o = b1o_ref[...]
    h1a = jnp.maximum(psa + pda + b1o[:, :CP], 0.0)
    h1b = jnp.maximum(psb + pdb + b1o[:, CP:], 0.0)
    w2o = w2o_ref[...]
    t = dot(h1a.astype(bf), w2o[:, :CP].astype(bf))
    t = t + dot(h1b.astype(bf), w2o[:, CP:].astype(bf))
    h2 = jnp.maximum(dot(e.astype(bf), w1t_ref[...].astype(bf))
                     + b1t_ref[...], 0.0)
    t = t + dot(h2.astype(bf), w2t_ref[...].astype(bf))
    hg = dot(jnp.maximum(dot(g_ref[...], w1g_ref[...]) + b1g_ref[...], 0.0),
             w2g_ref[...]) + b2g_ref[...]
    s = t + b2o_ref[...] + b2t_ref[...] + hg
    m = jnp.mean(s, axis=1, keepdims=True)
    v = jnp.mean((s - m) ** 2, axis=1, keepdims=True)
    sn = (s - m) * lax.rsqrt(v + 1e-5)
    o_ref[...] = e + jnp.maximum(sn, 0.0)


def _main_first_body(*refs):
    _main_body(None, *refs)


def _main_seg(k, canvas, ps, pd, edge, W2_one, W1_two, W2_two,
              b1o, b1t, b2o, b2t, g, W1_three, b1g, W2_three, b2g):
    full = lambda: pl.BlockSpec((C, C), lambda i: (0, 0))
    vec = lambda: pl.BlockSpec((1, C), lambda i: (0, 0))
    koff = k * (E_SEG // BLK_E)
    in_specs = [
        pl.BlockSpec((BLK_E, CP), lambda i: (i, 0)),
        pl.BlockSpec((BLK_E, CP), lambda i: (i, 0)),
        pl.BlockSpec((BLK_E, C), lambda i, kk=koff: (i + kk, 0)),
        full(), full(), full(),
        vec(), vec(), vec(), vec(),
        vec(), full(), vec(), full(), vec(),
    ]
    args = (ps, pd, edge, W2_one, W1_two, W2_two, b1o, b1t, b2o, b2t,
            g, W1_three, b1g, W2_three, b2g)
    if canvas is None:
        body, alias = _main_first_body, {}
    else:
        body, alias = _main_body, {0: 0}
        in_specs = [pl.BlockSpec(memory_space=pl.ANY)] + in_specs
        args = (canvas,) + args
    return pl.pallas_call(
        body,
        grid=(E_SEG // BLK_E,),
        in_specs=in_specs,
        out_specs=pl.BlockSpec((BLK_E, C), lambda i, kk=koff: (i + kk, 0)),
        out_shape=jax.ShapeDtypeStruct((N_EDGES, C), jnp.float32),
        input_output_aliases=alias,
    )(*args)


def kernel(node_features, edge_index, edge_features, global_features,
           W1_one, b1_one, W2_one, b2_one,
           W1_two, b1_two, W2_two, b2_two,
           W1_three, b1_three, W2_three, b2_three):
    node = node_features[0]
    edge = edge_features[0]
    src = edge_index[0, 0]
    dst = edge_index[0, 1]
    zp = jnp.zeros((E_PAD_SEG - E_SEG,), jnp.int32)
    zt = jnp.zeros((IDX_PAD,), jnp.int32)

    P = _node_proj(node, W1_one)

    r2 = lambda b: b.reshape(1, C)
    acc = None
    for k in range(SEG):
        sl = slice(k * E_SEG, (k + 1) * E_SEG)
        idx_k = jnp.concatenate([src[sl], zp, dst[sl], zp, zt])
        ps, pd = _sc_gather()(P, idx_k)
        ps = ps.reshape(E_PAD_SEG, CP)
        pd = pd.reshape(E_PAD_SEG, CP)
        acc = _main_seg(k, acc, ps, pd, edge, W2_one, W1_two, W2_two,
                        r2(b1_one), r2(b1_two), r2(b2_one), r2(b2_two),
                        global_features[0], W1_three, r2(b1_three),
                        W2_three, r2(b2_three))
    return lax.stop_gradient(acc[None])


# speedup vs baseline: 1.0288x; 1.0288x over previous
import functools

import jax
import jax.numpy as jnp
from jax import lax
from jax.experimental import pallas as pl
from jax.experimental.pallas import tpu as pltpu
from jax.experimental.pallas import tpu_sc as plsc

N_NODES = 10000
N_EDGES = 160000
C = 256

NC = 2
NS = 16
NW = NC * NS

CP = 128
CH = 128
RING = 4

SEG = 4
E_SEG = N_EDGES // SEG
E_PAD_SEG = 40960
G = 2
RING2 = 2
NCHS = 2 * E_PAD_SEG // (CH * G) // NW
FAST_C = 0
NF = 10
NSL = 10
IDX_PAD = 16 * G * CH

BLK_N = 1000
BLK_E = 2000


def _proj_body(n_ref, w_ref, o_ref):
    p = lax.dot_general(n_ref[...], w_ref[...],
                        (((1,), (1,)), ((), ())),
                        preferred_element_type=jnp.float32)
    u = lax.bitcast_convert_type(p, jnp.uint32)
    r = u + jnp.uint32(0x7FFF) + ((u >> 16) & jnp.uint32(1))
    h = r >> 16
    packed = h[:, :CP] | (h[:, CP:] << 16)
    o_ref[...] = lax.bitcast_convert_type(packed, jnp.float32)


def _node_proj(node, W1_one):
    return pl.pallas_call(
        _proj_body,
        grid=(N_NODES // BLK_N,),
        in_specs=[
            pl.BlockSpec((BLK_N, C), lambda i: (i, 0)),
            pl.BlockSpec((C, C), lambda i: (0, 0)),
        ],
        out_specs=pl.BlockSpec((BLK_N, CP), lambda i: (i, 0)),
        out_shape=jax.ShapeDtypeStruct((N_NODES, CP), jnp.float32),
    )(node, W1_one)


def _sc_gather_body(table, idx2, out_s, out_d, idx_v, rows,
                    sg0, sg1, sw0, sw1):
    sg = [sg0, sg1]
    sw = [sw0, sw1]
    c = lax.axis_index("c")
    s = lax.axis_index("s")
    wid = s * NC + c
    is_src = wid < NS
    lw = jnp.where(is_src, wid, wid - NS)

    dps = G * CH
    nda = E_PAD_SEG // dps
    nd = jnp.where(c == FAST_C, NF, NSL)
    nd_even = NF if FAST_C == 0 else NSL
    pair = lw // 2
    base_arr = pair * (NF + NSL) + jnp.where(c == 0, 0, nd_even)
    desc0 = jnp.where(is_src, base_arr, nda + base_arr)

    pltpu.sync_copy(idx2.at[pl.ds(pl.multiple_of(desc0 * dps, 8), NF * dps)],
                    idx_v)

    def gcopy(t, b):
        return pltpu.make_async_copy(
            table.at[idx_v.at[pl.ds(t * dps, dps)]], rows.at[b], sg[b])

    def run(oref):
        def wcopy(t, b):
            return pltpu.make_async_copy(rows.at[b], oref.at[base_arr + t],
                                         sw[b])

        for b in range(RING2):
            gcopy(b, b).start()

        def outer(r, carry):
            t0 = r * RING2
            for b in range(RING2):
                t = t0 + b
                gcopy(t, b).wait()
                wcopy(t, b).start()
                pb = (b - 1) % RING2
                tn = t + RING2 - 1

                @pl.when(jnp.logical_and(t >= 1, tn < nd))
                def _():
                    wcopy(t - 1, pb).wait()
                    gcopy(tn, pb).start()
            return carry

        lax.fori_loop(0, nd // RING2, outer, 0)
        for b in range(RING2):
            wcopy(nd - RING2 + b, b).wait()

    @pl.when(is_src)
    def _():
        run(out_s)

    @pl.when(jnp.logical_not(is_src))
    def _():
        run(out_d)


@functools.cache
def _sc_gather():
    sr = E_PAD_SEG // (G * CH)
    return pl.kernel(
        _sc_gather_body,
        out_type=(jax.ShapeDtypeStruct((sr, G * CH, CP), jnp.float32),
                  jax.ShapeDtypeStruct((sr, G * CH, CP), jnp.float32)),
        mesh=plsc.VectorSubcoreMesh(core_axis_name="c", subcore_axis_name="s",
                                    num_cores=NC, num_subcores=NS),
        scratch_types=[
            pltpu.VMEM((NF * G * CH,), jnp.int32),
            pltpu.VMEM((RING2, G * CH, CP), jnp.float32),
        ] + [pltpu.SemaphoreType.DMA] * 4,
    )


def _main_body(canvas_ref, ps_ref, pd_ref, e_ref, w2o_ref, w1t_ref, w2t_ref,
               b1o_ref, b1t_ref, b2o_ref, b2t_ref,
               g_ref, w1g_ref, b1g_ref, w2g_ref, b2g_ref, o_ref):
    del canvas_ref
    dot = lambda a, b: lax.dot_general(a, b, (((1,), (1,)), ((), ())),
                                       preferred_element_type=jnp.float32)

    def unpack(ref):
        w = lax.bitcast_convert_type(ref[...], jnp.uint32)
        lo = lax.bitcast_convert_type(w << 16, jnp.float32)
        hi = lax.bitcast_convert_type(w & jnp.uint32(0xFFFF0000), jnp.float32)
        return lo, hi

    bf = jnp.bfloat16
    e = e_ref[...]
    psa, psb = unpack(ps_ref)
    pda, pdb = unpack(pd_ref)
    b1o = b1o_ref[...]
    h1a = jnp.maximum(psa + pda + b1o[:, :CP], 0.0)
    h1b = jnp.maximum(psb + pdb + b1o[:, CP:], 0.0)
    w2o = w2o_ref[...]
    t = dot(h1a.astype(bf), w2o[:, :CP].astype(bf))
    t = t + dot(h1b.astype(bf), w2o[:, CP:].astype(bf))
    h2 = jnp.maximum(dot(e.astype(bf), w1t_ref[...].astype(bf))
                     + b1t_ref[...], 0.0)
    t = t + dot(h2.astype(bf), w2t_ref[...].astype(bf))
    hg = dot(jnp.maximum(dot(g_ref[...], w1g_ref[...]) + b1g_ref[...], 0.0),
             w2g_ref[...]) + b2g_ref[...]
    s = t + b2o_ref[...] + b2t_ref[...] + hg
    m = jnp.mean(s, axis=1, keepdims=True)
    v = jnp.mean((s - m) ** 2, axis=1, keepdims=True)
    sn = (s - m) * lax.rsqrt(v + 1e-5)
    o_ref[...] = e + jnp.maximum(sn, 0.0)


def _main_first_body(*refs):
    _main_body(None, *refs)


def _main_seg(k, canvas, ps, pd, edge, W2_one, W1_two, W2_two,
              b1o, b1t, b2o, b2t, g, W1_three, b1g, W2_three, b2g):
    full = lambda: pl.BlockSpec((C, C), lambda i: (0, 0))
    vec = lambda: pl.BlockSpec((1, C), lambda i: (0, 0))
    koff = k * (E_SEG // BLK_E)
    in_specs = [
        pl.BlockSpec((BLK_E, CP), lambda i: (i, 0)),
        pl.BlockSpec((BLK_E, CP), lambda i: (i, 0)),
        pl.BlockSpec((BLK_E, C), lambda i, kk=koff: (i + kk, 0)),
        full(), full(), full(),
        vec(), vec(), vec(), vec(),
        vec(), full(), vec(), full(), vec(),
    ]
    args = (ps, pd, edge, W2_one, W1_two, W2_two, b1o, b1t, b2o, b2t,
            g, W1_three, b1g, W2_three, b2g)
    if canvas is None:
        body, alias = _main_first_body, {}
    else:
        body, alias = _main_body, {0: 0}
        in_specs = [pl.BlockSpec(memory_space=pl.ANY)] + in_specs
        args = (canvas,) + args
    return pl.pallas_call(
        body,
        grid=(E_SEG // BLK_E,),
        in_specs=in_specs,
        out_specs=pl.BlockSpec((BLK_E, C), lambda i, kk=koff: (i + kk, 0)),
        out_shape=jax.ShapeDtypeStruct((N_EDGES, C), jnp.float32),
        input_output_aliases=alias,
    )(*args)


def kernel(node_features, edge_index, edge_features, global_features,
           W1_one, b1_one, W2_one, b2_one,
           W1_two, b1_two, W2_two, b2_two,
           W1_three, b1_three, W2_three, b2_three):
    node = node_features[0]
    edge = edge_features[0]
    src = edge_index[0, 0]
    dst = edge_index[0, 1]
    zp = jnp.zeros((E_PAD_SEG - E_SEG,), jnp.int32)
    zt = jnp.zeros((IDX_PAD,), jnp.int32)

    P = _node_proj(node, W1_one)

    r2 = lambda b: b.reshape(1, C)
    acc = None
    for k in range(SEG):
        sl = slice(k * E_SEG, (k + 1) * E_SEG)
        idx_k = jnp.concatenate([src[sl], zp, dst[sl], zp, zt])
        ps, pd = _sc_gather()(P, idx_k)
        ps = ps.reshape(E_PAD_SEG, CP)
        pd = pd.reshape(E_PAD_SEG, CP)
        acc = _main_seg(k, acc, ps, pd, edge, W2_one, W1_two, W2_two,
                        r2(b1_one), r2(b1_two), r2(b2_one), r2(b2_two),
                        global_features[0], W1_three, r2(b1_three),
                        W2_three, r2(b2_three))
    return lax.stop_gradient(acc[None])


# speedup vs baseline: 1.0425x; 1.0134x over previous
import functools

import jax
import jax.numpy as jnp
from jax import lax
from jax.experimental import pallas as pl
from jax.experimental.pallas import tpu as pltpu
from jax.experimental.pallas import tpu_sc as plsc

N_NODES = 10000
N_EDGES = 160000
C = 256

NC = 2
NS = 16
NW = NC * NS

CP = 128
CH = 128
RING = 4

SEG = 4
E_SEG = N_EDGES // SEG
E_PAD_SEG = 40960
G = 1
RING2 = 4
NCHS = 2 * E_PAD_SEG // (CH * G) // NW
FAST_C = 0
NF = 20
NSL = 20
IDX_PAD = 16 * G * CH

BLK_N = 1000
BLK_E = 2000


def _proj_body(n_ref, w_ref, o_ref):
    p = lax.dot_general(n_ref[...], w_ref[...],
                        (((1,), (1,)), ((), ())),
                        preferred_element_type=jnp.float32)
    u = lax.bitcast_convert_type(p, jnp.uint32)
    r = u + jnp.uint32(0x7FFF) + ((u >> 16) & jnp.uint32(1))
    h = r >> 16
    packed = h[:, :CP] | (h[:, CP:] << 16)
    o_ref[...] = lax.bitcast_convert_type(packed, jnp.float32)


def _node_proj(node, W1_one):
    return pl.pallas_call(
        _proj_body,
        grid=(N_NODES // BLK_N,),
        in_specs=[
            pl.BlockSpec((BLK_N, C), lambda i: (i, 0)),
            pl.BlockSpec((C, C), lambda i: (0, 0)),
        ],
        out_specs=pl.BlockSpec((BLK_N, CP), lambda i: (i, 0)),
        out_shape=jax.ShapeDtypeStruct((N_NODES, CP), jnp.float32),
    )(node, W1_one)


def _sc_gather_body(table, idx2, out_s, out_d, idx_v, rows,
                    sg0, sg1, sg2, sg3, sw0, sw1, sw2, sw3):
    sg = [sg0, sg1, sg2, sg3]
    sw = [sw0, sw1, sw2, sw3]
    c = lax.axis_index("c")
    s = lax.axis_index("s")
    wid = s * NC + c
    is_src = wid < NS
    lw = jnp.where(is_src, wid, wid - NS)

    dps = G * CH
    nda = E_PAD_SEG // dps
    nd = jnp.where(c == FAST_C, NF, NSL)
    nd_even = NF if FAST_C == 0 else NSL
    pair = lw // 2
    base_arr = pair * (NF + NSL) + jnp.where(c == 0, 0, nd_even)
    desc0 = jnp.where(is_src, base_arr, nda + base_arr)

    pltpu.sync_copy(idx2.at[pl.ds(pl.multiple_of(desc0 * dps, 8), NF * dps)],
                    idx_v)

    def gcopy(t, b):
        return pltpu.make_async_copy(
            table.at[idx_v.at[pl.ds(t * dps, dps)]], rows.at[b], sg[b])

    def run(oref):
        def wcopy(t, b):
            return pltpu.make_async_copy(rows.at[b], oref.at[base_arr + t],
                                         sw[b])

        for b in range(RING2):
            gcopy(b, b).start()

        def outer(r, carry):
            t0 = r * RING2
            for b in range(RING2):
                t = t0 + b
                gcopy(t, b).wait()
                wcopy(t, b).start()
                pb = (b - 1) % RING2
                tn = t + RING2 - 1

                @pl.when(jnp.logical_and(t >= 1, tn < nd))
                def _():
                    wcopy(t - 1, pb).wait()
                    gcopy(tn, pb).start()
            return carry

        lax.fori_loop(0, nd // RING2, outer, 0)
        for b in range(RING2):
            wcopy(nd - RING2 + b, b).wait()

    @pl.when(is_src)
    def _():
        run(out_s)

    @pl.when(jnp.logical_not(is_src))
    def _():
        run(out_d)


@functools.cache
def _sc_gather():
    sr = E_PAD_SEG // (G * CH)
    return pl.kernel(
        _sc_gather_body,
        out_type=(jax.ShapeDtypeStruct((sr, G * CH, CP), jnp.float32),
                  jax.ShapeDtypeStruct((sr, G * CH, CP), jnp.float32)),
        mesh=plsc.VectorSubcoreMesh(core_axis_name="c", subcore_axis_name="s",
                                    num_cores=NC, num_subcores=NS),
        scratch_types=[
            pltpu.VMEM((NF * G * CH,), jnp.int32),
            pltpu.VMEM((RING2, G * CH, CP), jnp.float32),
        ] + [pltpu.SemaphoreType.DMA] * (2 * RING2),
    )


def _main_body(canvas_ref, ps_ref, pd_ref, e_ref, w2o_ref, w1t_ref, w2t_ref,
               b1o_ref, b1t_ref, b2o_ref, b2t_ref,
               g_ref, w1g_ref, b1g_ref, w2g_ref, b2g_ref, o_ref):
    del canvas_ref
    dot = lambda a, b: lax.dot_general(a, b, (((1,), (1,)), ((), ())),
                                       preferred_element_type=jnp.float32)

    def unpack(ref):
        w = lax.bitcast_convert_type(ref[...], jnp.uint32)
        lo = lax.bitcast_convert_type(w << 16, jnp.float32)
        hi = lax.bitcast_convert_type(w & jnp.uint32(0xFFFF0000), jnp.float32)
        return lo, hi

    bf = jnp.bfloat16
    e = e_ref[...]
    psa, psb = unpack(ps_ref)
    pda, pdb = unpack(pd_ref)
    b1o = b1o_ref[...]
    h1a = jnp.maximum(psa + pda + b1o[:, :CP], 0.0)
    h1b = jnp.maximum(psb + pdb + b1o[:, CP:], 0.0)
    w2o = w2o_ref[...]
    t = dot(h1a.astype(bf), w2o[:, :CP].astype(bf))
    t = t + dot(h1b.astype(bf), w2o[:, CP:].astype(bf))
    h2 = jnp.maximum(dot(e.astype(bf), w1t_ref[...].astype(bf))
                     + b1t_ref[...], 0.0)
    t = t + dot(h2.astype(bf), w2t_ref[...].astype(bf))
    hg = dot(jnp.maximum(dot(g_ref[...], w1g_ref[...]) + b1g_ref[...], 0.0),
             w2g_ref[...]) + b2g_ref[...]
    s = t + b2o_ref[...] + b2t_ref[...] + hg
    m = jnp.mean(s, axis=1, keepdims=True)
    v = jnp.mean((s - m) ** 2, axis=1, keepdims=True)
    sn = (s - m) * lax.rsqrt(v + 1e-5)
    o_ref[...] = e + jnp.maximum(sn, 0.0)


def _main_first_body(*refs):
    _main_body(None, *refs)


def _main_seg(k, canvas, ps, pd, edge, W2_one, W1_two, W2_two,
              b1o, b1t, b2o, b2t, g, W1_three, b1g, W2_three, b2g):
    full = lambda: pl.BlockSpec((C, C), lambda i: (0, 0))
    vec = lambda: pl.BlockSpec((1, C), lambda i: (0, 0))
    koff = k * (E_SEG // BLK_E)
    in_specs = [
        pl.BlockSpec((BLK_E, CP), lambda i: (i, 0)),
        pl.BlockSpec((BLK_E, CP), lambda i: (i, 0)),
        pl.BlockSpec((BLK_E, C), lambda i, kk=koff: (i + kk, 0)),
        full(), full(), full(),
        vec(), vec(), vec(), vec(),
        vec(), full(), vec(), full(), vec(),
    ]
    args = (ps, pd, edge, W2_one, W1_two, W2_two, b1o, b1t, b2o, b2t,
            g, W1_three, b1g, W2_three, b2g)
    if canvas is None:
        body, alias = _main_first_body, {}
    else:
        body, alias = _main_body, {0: 0}
        in_specs = [pl.BlockSpec(memory_space=pl.ANY)] + in_specs
        args = (canvas,) + args
    return pl.pallas_call(
        body,
        grid=(E_SEG // BLK_E,),
        in_specs=in_specs,
        out_specs=pl.BlockSpec((BLK_E, C), lambda i, kk=koff: (i + kk, 0)),
        out_shape=jax.ShapeDtypeStruct((N_EDGES, C), jnp.float32),
        input_output_aliases=alias,
    )(*args)


def kernel(node_features, edge_index, edge_features, global_features,
           W1_one, b1_one, W2_one, b2_one,
           W1_two, b1_two, W2_two, b2_two,
           W1_three, b1_three, W2_three, b2_three):
    node = node_features[0]
    edge = edge_features[0]
    src = edge_index[0, 0]
    dst = edge_index[0, 1]
    zp = jnp.zeros((E_PAD_SEG - E_SEG,), jnp.int32)
    zt = jnp.zeros((IDX_PAD,), jnp.int32)

    P = _node_proj(node, W1_one)

    r2 = lambda b: b.reshape(1, C)
    acc = None
    for k in range(SEG):
        sl = slice(k * E_SEG, (k + 1) * E_SEG)
        idx_k = jnp.concatenate([src[sl], zp, dst[sl], zp, zt])
        ps, pd = _sc_gather()(P, idx_k)
        ps = ps.reshape(E_PAD_SEG, CP)
        pd = pd.reshape(E_PAD_SEG, CP)
        acc = _main_seg(k, acc, ps, pd, edge, W2_one, W1_two, W2_two,
                        r2(b1_one), r2(b1_two), r2(b2_one), r2(b2_two),
                        global_features[0], W1_three, r2(b1_three),
                        W2_three, r2(b2_three))
    return lax.stop_gradient(acc[None])


# speedup vs baseline: 1.0444x; 1.0019x over previous
import functools

import jax
import jax.numpy as jnp
from jax import lax
from jax.experimental import pallas as pl
from jax.experimental.pallas import tpu as pltpu
from jax.experimental.pallas import tpu_sc as plsc

N_NODES = 10000
N_EDGES = 160000
C = 256

NC = 2
NS = 16
NW = NC * NS

CP = 128
CH = 128

SEG = 4
E_SEG = N_EDGES // SEG
E_PAD_SEG = 40960
G = 1
RING2 = 4
NCHS = 2 * E_PAD_SEG // (CH * G) // NW
FAST_C = 0
NF = 20
NSL = 20
IDX_PAD = 16 * G * CH

BLK_N = 1000
BLK_E = 2000


def _proj_body(n_ref, w_ref, o_ref):
    p = lax.dot_general(n_ref[...], w_ref[...],
                        (((1,), (1,)), ((), ())),
                        preferred_element_type=jnp.float32)
    u = lax.bitcast_convert_type(p, jnp.uint32)
    r = u + jnp.uint32(0x7FFF) + ((u >> 16) & jnp.uint32(1))
    h = r >> 16
    packed = h[:, :CP] | (h[:, CP:] << 16)
    o_ref[...] = lax.bitcast_convert_type(packed, jnp.float32)


def _node_proj(node, W1_one):
    return pl.pallas_call(
        _proj_body,
        grid=(N_NODES // BLK_N,),
        in_specs=[
            pl.BlockSpec((BLK_N, C), lambda i: (i, 0)),
            pl.BlockSpec((C, C), lambda i: (0, 0)),
        ],
        out_specs=pl.BlockSpec((BLK_N, CP), lambda i: (i, 0)),
        out_shape=jax.ShapeDtypeStruct((N_NODES, CP), jnp.float32),
    )(node, W1_one)


def _sc_gather_body(table, idx2, out_s, out_d, idx_v, rows,
                    sg0, sg1, sg2, sg3, sw0, sw1, sw2, sw3):
    sg = [sg0, sg1, sg2, sg3]
    sw = [sw0, sw1, sw2, sw3]
    c = lax.axis_index("c")
    s = lax.axis_index("s")
    wid = s * NC + c
    is_src = wid < NS
    lw = jnp.where(is_src, wid, wid - NS)

    dps = G * CH
    nda = E_PAD_SEG // dps
    nd = jnp.where(c == FAST_C, NF, NSL)
    nd_even = NF if FAST_C == 0 else NSL
    pair = lw // 2
    base_arr = pair * (NF + NSL) + jnp.where(c == 0, 0, nd_even)
    desc0 = jnp.where(is_src, base_arr, nda + base_arr)

    pltpu.sync_copy(idx2.at[pl.ds(pl.multiple_of(desc0 * dps, 8), NF * dps)],
                    idx_v)

    def gcopy(t, b):
        return pltpu.make_async_copy(
            table.at[idx_v.at[pl.ds(t * dps, dps)]], rows.at[b], sg[b])

    def run(oref):
        def wcopy(t, b):
            return pltpu.make_async_copy(rows.at[b], oref.at[base_arr + t],
                                         sw[b])

        for b in range(RING2):
            gcopy(b, b).start()

        def outer(r, carry):
            t0 = r * RING2
            for b in range(RING2):
                t = t0 + b
                gcopy(t, b).wait()
                wcopy(t, b).start()
                pb = (b - 1) % RING2
                tn = t + RING2 - 1

                @pl.when(jnp.logical_and(t >= 1, tn < nd))
                def _():
                    wcopy(t - 1, pb).wait()
                    gcopy(tn, pb).start()
            return carry

        lax.fori_loop(0, nd // RING2, outer, 0)
        for b in range(RING2):
            wcopy(nd - RING2 + b, b).wait()

    @pl.when(is_src)
    def _():
        run(out_s)

    @pl.when(jnp.logical_not(is_src))
    def _():
        run(out_d)


@functools.cache
def _sc_gather():
    sr = E_PAD_SEG // (G * CH)
    return pl.kernel(
        _sc_gather_body,
        out_type=(jax.ShapeDtypeStruct((sr, G * CH, CP), jnp.float32),
                  jax.ShapeDtypeStruct((sr, G * CH, CP), jnp.float32)),
        mesh=plsc.VectorSubcoreMesh(core_axis_name="c", subcore_axis_name="s",
                                    num_cores=NC, num_subcores=NS),
        scratch_types=[
            pltpu.VMEM((NF * G * CH,), jnp.int32),
            pltpu.VMEM((RING2, G * CH, CP), jnp.float32),
        ] + [pltpu.SemaphoreType.DMA] * (2 * RING2),
    )


def _main_body(canvas_ref, ps_ref, pd_ref, e_ref, w2o_ref, w1t_ref, w2t_ref,
               b1o_ref, b1t_ref, b2o_ref, b2t_ref,
               g_ref, w1g_ref, b1g_ref, w2g_ref, b2g_ref, o_ref):
    del canvas_ref
    dot = lambda a, b: lax.dot_general(a, b, (((1,), (1,)), ((), ())),
                                       preferred_element_type=jnp.float32)

    def unpack(ref):
        w = lax.bitcast_convert_type(ref[...], jnp.uint32)
        lo = lax.bitcast_convert_type(w << 16, jnp.float32)
        hi = lax.bitcast_convert_type(w & jnp.uint32(0xFFFF0000), jnp.float32)
        return lo, hi

    bf = jnp.bfloat16
    e = e_ref[...]
    psa, psb = unpack(ps_ref)
    pda, pdb = unpack(pd_ref)
    b1o = b1o_ref[...]
    h1a = jnp.maximum(psa + pda + b1o[:, :CP], 0.0)
    h1b = jnp.maximum(psb + pdb + b1o[:, CP:], 0.0)
    w2o = w2o_ref[...]
    t = dot(h1a.astype(bf), w2o[:, :CP].astype(bf))
    t = t + dot(h1b.astype(bf), w2o[:, CP:].astype(bf))
    h2 = jnp.maximum(dot(e.astype(bf), w1t_ref[...].astype(bf))
                     + b1t_ref[...], 0.0)
    t = t + dot(h2.astype(bf), w2t_ref[...].astype(bf))
    hg = dot(jnp.maximum(dot(g_ref[...], w1g_ref[...]) + b1g_ref[...], 0.0),
             w2g_ref[...]) + b2g_ref[...]
    s = t + b2o_ref[...] + b2t_ref[...] + hg
    m = jnp.mean(s, axis=1, keepdims=True)
    v = jnp.mean((s - m) ** 2, axis=1, keepdims=True)
    sn = (s - m) * lax.rsqrt(v + 1e-5)
    o_ref[...] = e + jnp.maximum(sn, 0.0)


def _main_first_body(*refs):
    _main_body(None, *refs)


def _main_seg(k, canvas, ps, pd, edge, W2_one, W1_two, W2_two,
              b1o, b1t, b2o, b2t, g, W1_three, b1g, W2_three, b2g):
    full = lambda: pl.BlockSpec((C, C), lambda i: (0, 0))
    vec = lambda: pl.BlockSpec((1, C), lambda i: (0, 0))
    koff = k * (E_SEG // BLK_E)
    in_specs = [
        pl.BlockSpec((BLK_E, CP), lambda i: (i, 0)),
        pl.BlockSpec((BLK_E, CP), lambda i: (i, 0)),
        pl.BlockSpec((BLK_E, C), lambda i, kk=koff: (i + kk, 0)),
        full(), full(), full(),
        vec(), vec(), vec(), vec(),
        vec(), full(), vec(), full(), vec(),
    ]
    args = (ps, pd, edge, W2_one, W1_two, W2_two, b1o, b1t, b2o, b2t,
            g, W1_three, b1g, W2_three, b2g)
    if canvas is None:
        body, alias = _main_first_body, {}
    else:
        body, alias = _main_body, {0: 0}
        in_specs = [pl.BlockSpec(memory_space=pl.ANY)] + in_specs
        args = (canvas,) + args
    return pl.pallas_call(
        body,
        grid=(E_SEG // BLK_E,),
        in_specs=in_specs,
        out_specs=pl.BlockSpec((BLK_E, C), lambda i, kk=koff: (i + kk, 0)),
        out_shape=jax.ShapeDtypeStruct((N_EDGES, C), jnp.float32),
        input_output_aliases=alias,
    )(*args)


def kernel(node_features, edge_index, edge_features, global_features,
           W1_one, b1_one, W2_one, b2_one,
           W1_two, b1_two, W2_two, b2_two,
           W1_three, b1_three, W2_three, b2_three):
    node = node_features[0]
    edge = edge_features[0]
    src = edge_index[0, 0]
    dst = edge_index[0, 1]
    zp = jnp.zeros((E_PAD_SEG - E_SEG,), jnp.int32)
    zt = jnp.zeros((IDX_PAD,), jnp.int32)

    P = _node_proj(node, W1_one)

    r2 = lambda b: b.reshape(1, C)
    acc = None
    for k in range(SEG):
        sl = slice(k * E_SEG, (k + 1) * E_SEG)
        idx_k = jnp.concatenate([src[sl], zp, dst[sl], zp, zt])
        ps, pd = _sc_gather()(P, idx_k)
        ps = ps.reshape(E_PAD_SEG, CP)
        pd = pd.reshape(E_PAD_SEG, CP)
        acc = _main_seg(k, acc, ps, pd, edge, W2_one, W1_two, W2_two,
                        r2(b1_one), r2(b1_two), r2(b2_one), r2(b2_two),
                        global_features[0], W1_three, r2(b1_three),
                        W2_three, r2(b2_three))
    return lax.stop_gradient(acc[None])
